# baseline (speedup 1.0000x reference)
import functools

import jax
import jax.numpy as jnp
from jax import lax
from jax.experimental import pallas as pl
from jax.experimental.pallas import tpu as pltpu
from jax.experimental.pallas import tpu_sc as plsc
from jax._src.pallas import mpmd as _mpmd

_NC = 2
_NS = 16
_NW = _NC * _NS
_L = 16
_CH = 128


@functools.cache
def _make_sc_update(m, d, b):
    per_w = m // _NW
    nvec = b // _L
    mesh = plsc.VectorSubcoreMesh(
        core_axis_name="c", subcore_axis_name="s", num_cores=_NC)

    def sc_update(mem_in, idx_hbm, val_hbm, out_hbm, idxb, rows_l, pos_l,
                  wp_l, aux, rstage, wstage, rowsbuf, gsem, ssem):
        del mem_in
        wid = lax.axis_index("s") * _NC + lax.axis_index("c")
        lo = wid * per_w
        hi = lo + per_w
        lane = lax.iota(jnp.int32, _L)

        pltpu.sync_copy(idx_hbm, idxb)

        def scan_body(t, cnt):
            v = idxb[pl.ds(t * _L, _L)]
            msk = (v >= lo) & (v < hi)
            plsc.store_compressed(rows_l.at[pl.ds(cnt, _L)], v, mask=msk)
            plsc.store_compressed(
                pos_l.at[pl.ds(cnt, _L)], t * _L + lane, mask=msk)
            return cnt + jnp.max(plsc.all_reduce_population_count(msk))

        cnt = lax.fori_loop(0, nvec, scan_body, jnp.int32(0))

        def _ld(ref, k):
            return ref[pl.ds(k, _L)][0]

        def dedup_body(k, _):
            rl = jnp.clip(_ld(rows_l, k) - lo, 0, per_w - 1)
            plsc.store_compressed(
                aux.at[pl.ds(rl, _L)],
                jnp.full((_L,), k, jnp.int32), mask=lane == 0)
            return _

        lax.fori_loop(0, cnt, dedup_body, jnp.int32(0))

        def win_body(t, _):
            off = t * _L
            rvec = rows_l[pl.ds(off, _L)]
            rloc = jnp.clip(rvec - lo, 0, per_w - 1)
            kw = jnp.clip(plsc.load_gather(aux, [rloc]), 0, b - 1)
            wp_l[pl.ds(off, _L)] = plsc.load_gather(pos_l, [kw])
            return _

        lax.fori_loop(0, (cnt + _L - 1) // _L, win_body, jnp.int32(0))

        @pl.when(cnt > 0)
        def _():
            r0 = jnp.full((_L,), _ld(rows_l, 0), jnp.int32)
            w0 = jnp.full((_L,), _ld(wp_l, 0), jnp.int32)
            for j in range(_CH // _L):
                rows_l[pl.ds(cnt + j * _L, _L)] = r0
                wp_l[pl.ds(cnt + j * _L, _L)] = w0

        def apply_body(c, _):
            for j in range(_CH // _L):
                off = c * _CH + j * _L
                rstage[pl.ds(j * _L, _L)] = jnp.clip(
                    rows_l[pl.ds(off, _L)], 0, m - 1)
                wstage[pl.ds(j * _L, _L)] = jnp.clip(
                    wp_l[pl.ds(off, _L)], 0, b - 1)
            pltpu.async_copy(val_hbm.at[wstage], rowsbuf, gsem).wait()
            pltpu.async_copy(rowsbuf, out_hbm.at[rstage], ssem).wait()
            return _

        lax.fori_loop(0, (cnt + _CH - 1) // _CH, apply_body, jnp.int32(0))

    return _mpmd._mpmd_map(
        [(mesh, sc_update)],
        jax.ShapeDtypeStruct((m, d), jnp.float32),
        input_output_aliases={0: 0},
        scratch_types=[
            pltpu.VMEM((b,), jnp.int32),
            pltpu.VMEM((b + _CH + _L,), jnp.int32),
            pltpu.VMEM((b + _CH + _L,), jnp.int32),
            pltpu.VMEM((b + _CH + _L,), jnp.int32),
            pltpu.VMEM((per_w + _L,), jnp.int32),
            pltpu.VMEM((_CH,), jnp.int32),
            pltpu.VMEM((_CH,), jnp.int32),
            pltpu.VMEM((_CH, d), jnp.float32),
            pltpu.SemaphoreType.DMA,
            pltpu.SemaphoreType.DMA,
        ],
        compiler_params=pltpu.CompilerParams(
            use_tc_tiling_on_sc=False, needs_layout_passes=False),
    )


def kernel(memory, idx, val):
    m, d = memory.shape
    b = idx.shape[0]
    return _make_sc_update(m, d, b)(memory, idx, val)
